# baseline (speedup 1.0000x reference)
import functools

import jax
import jax.numpy as jnp
from jax import lax
from jax.experimental import pallas as pl
from jax.experimental.pallas import tpu as pltpu
from jax.experimental.pallas import tpu_sc as plsc

N = 10000
NPAD = 10240
E = 320000
D = 128
D3 = 3 * D
NUM_LAYERS = 3
NC = 2
NS = 16
NW = NC * NS
CHUNK = 128
K = 80
EPAD = NW * K * CHUNK
ROWS_PER_TILE = NPAD // NS


def _idx_start(s_hbm, d_hbm, wid, j, s_buf, d_buf, sem):
    pltpu.async_copy(s_hbm.at[wid, j], s_buf, sem)
    pltpu.async_copy(d_hbm.at[wid, j], d_buf, sem)


def _idx_wait(s_hbm, d_hbm, wid, j, s_buf, d_buf, sem):
    pltpu.make_async_copy(s_hbm.at[wid, j], s_buf, sem).wait()
    pltpu.make_async_copy(d_hbm.at[wid, j], d_buf, sem).wait()


def _segsum_body(h_hbm, s_hbm, d_hbm, out_hbm, sa, da, sb, db, rows_a, rows_b,
                 agg_sh, isem_a, isem_b, gsem_a, gsem_b):
    ci = lax.axis_index("c")
    si = lax.axis_index("s")
    wid = si * NC + ci
    _idx_start(s_hbm, d_hbm, wid, 0, sa, da, isem_a)
    _idx_start(s_hbm, d_hbm, wid, 1, sb, db, isem_b)
    zero16 = jnp.zeros((16,), jnp.float32)

    def zbody(i, c):
        for j in range(D // 16):
            rows_a[i, pl.ds(j * 16, 16)] = zero16
        return c

    lax.fori_loop(0, CHUNK, zbody, 0)
    base = si * ROWS_PER_TILE
    for k in range(ROWS_PER_TILE // CHUNK):
        pltpu.sync_copy(rows_a, agg_sh.at[pl.ds(base + k * CHUNK, CHUNK)])
    _idx_wait(s_hbm, d_hbm, wid, 0, sa, da, isem_a)
    pltpu.async_copy(h_hbm.at[sa], rows_a, gsem_a)
    plsc.subcore_barrier()

    def ebody(p, c):
        j = 2 * p
        _idx_wait(s_hbm, d_hbm, wid, j + 1, sb, db, isem_b)
        pltpu.async_copy(h_hbm.at[sb], rows_b, gsem_b)
        pltpu.make_async_copy(h_hbm.at[sa], rows_a, gsem_a).wait()
        pltpu.sync_copy(rows_a, agg_sh.at[da], add=True)
        j2 = jnp.minimum(j + 2, K - 1)
        _idx_start(s_hbm, d_hbm, wid, j2, sa, da, isem_a)
        pltpu.make_async_copy(h_hbm.at[sb], rows_b, gsem_b).wait()
        _idx_wait(s_hbm, d_hbm, wid, j2, sa, da, isem_a)
        pltpu.async_copy(h_hbm.at[sa], rows_a, gsem_a)
        pltpu.sync_copy(rows_b, agg_sh.at[db], add=True)
        j3 = jnp.minimum(j + 3, K - 1)
        _idx_start(s_hbm, d_hbm, wid, j3, sb, db, isem_b)
        return c

    lax.fori_loop(0, K // 2, ebody, 0)
    pltpu.make_async_copy(h_hbm.at[sa], rows_a, gsem_a).wait()
    _idx_wait(s_hbm, d_hbm, wid, K - 1, sb, db, isem_b)
    plsc.subcore_barrier()
    pltpu.sync_copy(agg_sh.at[pl.ds(base, ROWS_PER_TILE)],
                    out_hbm.at[ci, pl.ds(base, ROWS_PER_TILE)])


_sc_segsum = functools.partial(
    pl.kernel,
    mesh=plsc.VectorSubcoreMesh(core_axis_name="c", subcore_axis_name="s"),
    out_type=jax.ShapeDtypeStruct((NC, NPAD, D), jnp.float32),
    scratch_types=[
        pltpu.VMEM((CHUNK,), jnp.int32),
        pltpu.VMEM((CHUNK,), jnp.int32),
        pltpu.VMEM((CHUNK,), jnp.int32),
        pltpu.VMEM((CHUNK,), jnp.int32),
        pltpu.VMEM((CHUNK, D), jnp.float32),
        pltpu.VMEM((CHUNK, D), jnp.float32),
        pltpu.VMEM_SHARED((NPAD, D), jnp.float32),
        pltpu.SemaphoreType.DMA,
        pltpu.SemaphoreType.DMA,
        pltpu.SemaphoreType.DMA,
        pltpu.SemaphoreType.DMA,
    ],
)(_segsum_body)

BN = 400


def _nt_body(x_ref, wt_ref, b_ref, o_ref):
    o_ref[...] = jnp.maximum(
        jnp.dot(x_ref[...], wt_ref[...], preferred_element_type=jnp.float32)
        + b_ref[...], 0.0)


def _node_transform(x, wt, b):
    return pl.pallas_call(
        _nt_body,
        grid=(N // BN,),
        in_specs=[
            pl.BlockSpec((BN, D), lambda i: (i, 0)),
            pl.BlockSpec((D, D), lambda i: (0, 0)),
            pl.BlockSpec((1, D), lambda i: (0, 0)),
        ],
        out_specs=pl.BlockSpec((BN, D), lambda i: (i, 0)),
        out_shape=jax.ShapeDtypeStruct((N, D), jnp.float32),
    )(x, wt, b)


def _gru_body(parts_ref, h_ref, wg_ref, wiht_ref, bih_ref, whht_ref, bhh_ref, o_ref):
    s = parts_ref[0] + parts_ref[1]
    h = h_ref[...]
    agg = jnp.dot(s, wg_ref[...], preferred_element_type=jnp.float32)
    gi = jnp.dot(agg, wiht_ref[...], preferred_element_type=jnp.float32) + bih_ref[...]
    gh = jnp.dot(h, whht_ref[...], preferred_element_type=jnp.float32) + bhh_ref[...]
    r = jax.nn.sigmoid(gi[:, 0:D] + gh[:, 0:D])
    z = jax.nn.sigmoid(gi[:, D:2 * D] + gh[:, D:2 * D])
    n = jnp.tanh(gi[:, 2 * D:] + r * gh[:, 2 * D:])
    o_ref[...] = (1.0 - z) * n + z * h


def _gru(parts, h, wg, wiht, bih, whht, bhh):
    return pl.pallas_call(
        _gru_body,
        grid=(N // BN,),
        in_specs=[
            pl.BlockSpec((NC, BN, D), lambda i: (0, i, 0)),
            pl.BlockSpec((BN, D), lambda i: (i, 0)),
            pl.BlockSpec((D, D), lambda i: (0, 0)),
            pl.BlockSpec((D, D3), lambda i: (0, 0)),
            pl.BlockSpec((1, D3), lambda i: (0, 0)),
            pl.BlockSpec((D, D3), lambda i: (0, 0)),
            pl.BlockSpec((1, D3), lambda i: (0, 0)),
        ],
        out_specs=pl.BlockSpec((BN, D), lambda i: (i, 0)),
        out_shape=jax.ShapeDtypeStruct((N, D), jnp.float32),
    )(parts, h, wg, wiht, bih, whht, bhh)


def kernel(x, edge_index, edge_attr, z_soft, W_nt, b_nt, W_ggc, W_ih, b_ih, W_hh, b_hh):
    del edge_attr, z_soft
    src = edge_index[0].astype(jnp.int32)
    dst = edge_index[1].astype(jnp.int32)
    pad = EPAD - E
    pad_idx = jnp.arange(pad, dtype=jnp.int32)
    pad_dst = N + pad_idx % (NPAD - N)
    pad_src = pad_idx % N
    src3 = jnp.concatenate([src, pad_src]).reshape(NW, K, CHUNK)
    dst3 = jnp.concatenate([dst, pad_dst]).reshape(NW, K, CHUNK)

    h = _node_transform(x, W_nt.T, b_nt.reshape(1, D))
    wiht = W_ih.T
    whht = W_hh.T
    bih = b_ih.reshape(1, D3)
    bhh = b_hh.reshape(1, D3)
    for i in range(NUM_LAYERS):
        parts = _sc_segsum(h, src3, dst3)
        h = _gru(parts, h, W_ggc[i], wiht, bih, whht, bhh)
    return h


# speedup vs baseline: 1.0980x; 1.0980x over previous
import functools

import jax
import jax.numpy as jnp
from jax import lax
from jax.experimental import pallas as pl
from jax.experimental.pallas import tpu as pltpu
from jax.experimental.pallas import tpu_sc as plsc

N = 10000
NPAD = 10240
E = 320000
D = 128
D3 = 3 * D
NUM_LAYERS = 3
NC = 2
NS = 16
NW = NC * NS
CHUNK = 128
K = 80
EPAD = NW * K * CHUNK
ROWS_PER_TILE = NPAD // NS


def _idx_start(s_hbm, d_hbm, wid, j, s_buf, d_buf, sem):
    pltpu.async_copy(s_hbm.at[wid, j], s_buf, sem)
    pltpu.async_copy(d_hbm.at[wid, j], d_buf, sem)


def _idx_wait(s_hbm, d_hbm, wid, j, s_buf, d_buf, sem):
    pltpu.make_async_copy(s_hbm.at[wid, j], s_buf, sem).wait()
    pltpu.make_async_copy(d_hbm.at[wid, j], d_buf, sem).wait()


def _segsum_body(h_hbm, s_hbm, d_hbm, out_hbm, sa, da, sb, db, rows_a, rows_b,
                 agg_sh, isem_a, isem_b, gsem_a, gsem_b):
    ci = lax.axis_index("c")
    si = lax.axis_index("s")
    wid = si * NC + ci
    _idx_start(s_hbm, d_hbm, wid, 0, sa, da, isem_a)
    _idx_start(s_hbm, d_hbm, wid, 1, sb, db, isem_b)
    zero16 = jnp.zeros((16,), jnp.float32)

    def zbody(i, c):
        for j in range(D // 16):
            rows_a[i, pl.ds(j * 16, 16)] = zero16
        return c

    lax.fori_loop(0, CHUNK, zbody, 0)
    base = si * ROWS_PER_TILE
    for k in range(ROWS_PER_TILE // CHUNK):
        pltpu.sync_copy(rows_a, agg_sh.at[pl.ds(base + k * CHUNK, CHUNK)])
    _idx_wait(s_hbm, d_hbm, wid, 0, sa, da, isem_a)
    pltpu.async_copy(h_hbm.at[sa], rows_a, gsem_a)
    plsc.subcore_barrier()

    def ebody(p, c):
        j = 2 * p
        _idx_wait(s_hbm, d_hbm, wid, j + 1, sb, db, isem_b)
        pltpu.async_copy(h_hbm.at[sb], rows_b, gsem_b)
        pltpu.make_async_copy(h_hbm.at[sa], rows_a, gsem_a).wait()
        pltpu.sync_copy(rows_a, agg_sh.at[da], add=True)
        j2 = jnp.minimum(j + 2, K - 1)
        _idx_start(s_hbm, d_hbm, wid, j2, sa, da, isem_a)
        pltpu.make_async_copy(h_hbm.at[sb], rows_b, gsem_b).wait()
        _idx_wait(s_hbm, d_hbm, wid, j2, sa, da, isem_a)
        pltpu.async_copy(h_hbm.at[sa], rows_a, gsem_a)
        pltpu.sync_copy(rows_b, agg_sh.at[db], add=True)
        j3 = jnp.minimum(j + 3, K - 1)
        _idx_start(s_hbm, d_hbm, wid, j3, sb, db, isem_b)
        return c

    lax.fori_loop(0, K // 2, ebody, 0)
    pltpu.make_async_copy(h_hbm.at[sa], rows_a, gsem_a).wait()
    _idx_wait(s_hbm, d_hbm, wid, K - 1, sb, db, isem_b)
    plsc.subcore_barrier()
    pltpu.sync_copy(agg_sh.at[pl.ds(base, ROWS_PER_TILE)],
                    out_hbm.at[ci, pl.ds(base, ROWS_PER_TILE)])


_sc_segsum = functools.partial(
    pl.kernel,
    mesh=plsc.VectorSubcoreMesh(core_axis_name="c", subcore_axis_name="s"),
    out_type=jax.ShapeDtypeStruct((NC, NPAD, D), jnp.float32),
    scratch_types=[
        pltpu.VMEM((CHUNK,), jnp.int32),
        pltpu.VMEM((CHUNK,), jnp.int32),
        pltpu.VMEM((CHUNK,), jnp.int32),
        pltpu.VMEM((CHUNK,), jnp.int32),
        pltpu.VMEM((CHUNK, D), jnp.float32),
        pltpu.VMEM((CHUNK, D), jnp.float32),
        pltpu.VMEM_SHARED((NPAD, D), jnp.float32),
        pltpu.SemaphoreType.DMA,
        pltpu.SemaphoreType.DMA,
        pltpu.SemaphoreType.DMA,
        pltpu.SemaphoreType.DMA,
    ],
)(_segsum_body)

BN = 2000


def _nt_body(x_ref, wt_ref, b_ref, o_ref):
    o_ref[...] = jnp.maximum(
        jnp.dot(x_ref[...], wt_ref[...], preferred_element_type=jnp.float32)
        + b_ref[...], 0.0)


def _node_transform(x, wt, b):
    return pl.pallas_call(
        _nt_body,
        grid=(N // BN,),
        in_specs=[
            pl.BlockSpec((BN, D), lambda i: (i, 0)),
            pl.BlockSpec((D, D), lambda i: (0, 0)),
            pl.BlockSpec((1, D), lambda i: (0, 0)),
        ],
        out_specs=pl.BlockSpec((BN, D), lambda i: (i, 0)),
        out_shape=jax.ShapeDtypeStruct((N, D), jnp.float32),
    )(x, wt, b)


def _gru_body(parts_ref, h_ref, wg_ref, wiht_ref, bih_ref, whht_ref, bhh_ref, o_ref):
    s = parts_ref[0] + parts_ref[1]
    h = h_ref[...]
    agg = jnp.dot(s, wg_ref[...], preferred_element_type=jnp.float32)
    gi = jnp.dot(agg, wiht_ref[...], preferred_element_type=jnp.float32) + bih_ref[...]
    gh = jnp.dot(h, whht_ref[...], preferred_element_type=jnp.float32) + bhh_ref[...]
    r = jax.nn.sigmoid(gi[:, 0:D] + gh[:, 0:D])
    z = jax.nn.sigmoid(gi[:, D:2 * D] + gh[:, D:2 * D])
    n = jnp.tanh(gi[:, 2 * D:] + r * gh[:, 2 * D:])
    o_ref[...] = (1.0 - z) * n + z * h


def _gru(parts, h, wg, wiht, bih, whht, bhh):
    return pl.pallas_call(
        _gru_body,
        grid=(N // BN,),
        in_specs=[
            pl.BlockSpec((NC, BN, D), lambda i: (0, i, 0)),
            pl.BlockSpec((BN, D), lambda i: (i, 0)),
            pl.BlockSpec((D, D), lambda i: (0, 0)),
            pl.BlockSpec((D, D3), lambda i: (0, 0)),
            pl.BlockSpec((1, D3), lambda i: (0, 0)),
            pl.BlockSpec((D, D3), lambda i: (0, 0)),
            pl.BlockSpec((1, D3), lambda i: (0, 0)),
        ],
        out_specs=pl.BlockSpec((BN, D), lambda i: (i, 0)),
        out_shape=jax.ShapeDtypeStruct((N, D), jnp.float32),
    )(parts, h, wg, wiht, bih, whht, bhh)


def kernel(x, edge_index, edge_attr, z_soft, W_nt, b_nt, W_ggc, W_ih, b_ih, W_hh, b_hh):
    del edge_attr, z_soft
    src = edge_index[0].astype(jnp.int32)
    dst = edge_index[1].astype(jnp.int32)
    pad = EPAD - E
    pad_idx = jnp.arange(pad, dtype=jnp.int32)
    pad_dst = N + pad_idx % (NPAD - N)
    pad_src = pad_idx % N
    src3 = jnp.concatenate([src, pad_src]).reshape(NW, K, CHUNK)
    dst3 = jnp.concatenate([dst, pad_dst]).reshape(NW, K, CHUNK)

    h = _node_transform(x, W_nt.T, b_nt.reshape(1, D))
    wiht = W_ih.T
    whht = W_hh.T
    bih = b_ih.reshape(1, D3)
    bhh = b_hh.reshape(1, D3)
    for i in range(NUM_LAYERS):
        parts = _sc_segsum(h, src3, dst3)
        h = _gru(parts, h, W_ggc[i], wiht, bih, whht, bhh)
    return h
